# baseline (speedup 1.0000x reference)
import functools

import jax
import jax.numpy as jnp
from jax import lax
from jax.experimental import pallas as pl
from jax.experimental.pallas import tpu as pltpu
from jax.experimental.pallas import tpu_sc as plsc

_K = 512
_D = 256
_B = 576
_BETA = 0.25
_TEMP = 0.1

_ROWS_PER_WORKER = 48
_CHUNK = _ROWS_PER_WORKER // 2
_NUM_WORKERS = _B // _ROWS_PER_WORKER


def _sc_gather(embedding, label):
    mesh = plsc.VectorSubcoreMesh(
        core_axis_name="c", subcore_axis_name="s", num_cores=1)

    @functools.partial(
        pl.kernel,
        mesh=mesh,
        out_type=jax.ShapeDtypeStruct((_B, _D), jnp.float32),
        scratch_types=[
            pltpu.VMEM((_CHUNK,), jnp.int32),
            pltpu.VMEM((_CHUNK,), jnp.int32),
            pltpu.VMEM((_CHUNK, _D), jnp.float32),
            pltpu.VMEM((_CHUNK, _D), jnp.float32),
            pltpu.SemaphoreType.DMA,
            pltpu.SemaphoreType.DMA,
            pltpu.SemaphoreType.DMA,
            pltpu.SemaphoreType.DMA,
        ],
    )
    def gather_kernel(table_hbm, idx_hbm, out_hbm, idx_a, idx_b, rows_a,
                      rows_b, sem_ga, sem_gb, sem_oa, sem_ob):
        wid = lax.axis_index("s")

        @pl.when(wid < _NUM_WORKERS)
        def _():
            base = wid * _ROWS_PER_WORKER
            pltpu.sync_copy(idx_hbm.at[pl.ds(base, _CHUNK)], idx_a)
            pltpu.sync_copy(idx_hbm.at[pl.ds(base + _CHUNK, _CHUNK)], idx_b)
            ga = pltpu.async_copy(table_hbm.at[idx_a], rows_a, sem_ga)
            gb = pltpu.async_copy(table_hbm.at[idx_b], rows_b, sem_gb)
            ga.wait()
            oa = pltpu.async_copy(
                rows_a, out_hbm.at[pl.ds(base, _CHUNK)], sem_oa)
            gb.wait()
            ob = pltpu.async_copy(
                rows_b, out_hbm.at[pl.ds(base + _CHUNK, _CHUNK)], sem_ob)
            oa.wait()
            ob.wait()

    return gather_kernel(embedding, label)


def _loss_kernel(x_ref, lab_ref, e_ref, loss_ref):
    x = x_ref[...]
    e = e_ref[...]
    lab = lab_ref[...]

    ohT = (jax.lax.broadcasted_iota(jnp.int32, (_K, _B), 0) == lab).astype(
        jnp.float32)
    p = jax.lax.dot_general(
        ohT, e, (((0,), (0,)), ((), ())),
        preferred_element_type=jnp.float32,
        precision=jax.lax.Precision.HIGHEST)

    mse = jnp.mean((p - x) ** 2)

    qn = x * jax.lax.rsqrt(jnp.sum(x * x, axis=1, keepdims=True))
    en = e * jax.lax.rsqrt(jnp.sum(e * e, axis=1, keepdims=True))
    pn = p * jax.lax.rsqrt(jnp.sum(p * p, axis=1, keepdims=True))

    logits = jax.lax.dot_general(
        qn, en, (((1,), (1,)), ((), ())),
        preferred_element_type=jnp.float32,
        precision=jax.lax.Precision.HIGHEST) * (1.0 / _TEMP)
    pos = jnp.sum(qn * pn, axis=1) * (1.0 / _TEMP)

    m = jnp.max(logits, axis=1)
    lse = jnp.log(jnp.sum(jnp.exp(logits - m[:, None]), axis=1)) + m
    infonce = jnp.mean(lse - pos)

    loss_ref[...] = jnp.reshape(mse * (1.0 + _BETA) + infonce, (1, 1))


def kernel(flat_latents, label, embedding):
    q = _sc_gather(embedding, label)
    loss = pl.pallas_call(
        _loss_kernel,
        out_shape=jax.ShapeDtypeStruct((1, 1), jnp.float32),
    )(flat_latents, label.reshape(1, _B), embedding)
    return q, loss.reshape(())


# speedup vs baseline: 1.0119x; 1.0119x over previous
import functools

import jax
import jax.numpy as jnp
from jax import lax
from jax.experimental import pallas as pl
from jax.experimental.pallas import tpu as pltpu
from jax.experimental.pallas import tpu_sc as plsc

_K = 512
_D = 256
_B = 576
_BETA = 0.25
_TEMP = 0.1

_ROWS_PER_WORKER = 48
_NUM_WORKERS = _B // _ROWS_PER_WORKER


def _sc_gather(embedding, label):
    mesh = plsc.VectorSubcoreMesh(
        core_axis_name="c", subcore_axis_name="s", num_cores=1)

    @functools.partial(
        pl.kernel,
        mesh=mesh,
        out_type=jax.ShapeDtypeStruct((_B, _D), jnp.float32),
        scratch_types=[
            pltpu.VMEM((_ROWS_PER_WORKER,), jnp.int32),
            pltpu.VMEM((_ROWS_PER_WORKER, _D), jnp.float32),
            pltpu.SemaphoreType.DMA,
        ],
    )
    def gather_kernel(table_hbm, idx_hbm, out_hbm, idx_v, rows_v, sem):
        wid = lax.axis_index("s")

        @pl.when(wid < _NUM_WORKERS)
        def _():
            base = wid * _ROWS_PER_WORKER
            pltpu.sync_copy(idx_hbm.at[pl.ds(base, _ROWS_PER_WORKER)], idx_v)
            pltpu.async_copy(table_hbm.at[idx_v], rows_v, sem).wait()
            pltpu.sync_copy(rows_v, out_hbm.at[pl.ds(base, _ROWS_PER_WORKER)])

    return gather_kernel(embedding, label)


def _loss_kernel(x_ref, lab_ref, e_ref, loss_ref):
    x = x_ref[...]
    e = e_ref[...]
    lab = lab_ref[...]

    ohT = (jax.lax.broadcasted_iota(jnp.int32, (_K, _B), 0) == lab).astype(
        jnp.float32)
    p = jax.lax.dot_general(
        ohT, e, (((0,), (0,)), ((), ())),
        preferred_element_type=jnp.float32,
        precision=jax.lax.Precision.HIGHEST)

    mse = jnp.mean((p - x) ** 2)

    qn = x * jax.lax.rsqrt(jnp.sum(x * x, axis=1, keepdims=True))
    en = e * jax.lax.rsqrt(jnp.sum(e * e, axis=1, keepdims=True))
    pn = p * jax.lax.rsqrt(jnp.sum(p * p, axis=1, keepdims=True))

    logits = jax.lax.dot_general(
        qn, en, (((1,), (1,)), ((), ())),
        preferred_element_type=jnp.float32,
        precision=jax.lax.Precision.HIGHEST) * (1.0 / _TEMP)
    pos = jnp.sum(qn * pn, axis=1) * (1.0 / _TEMP)

    m = jnp.max(logits, axis=1)
    lse = jnp.log(jnp.sum(jnp.exp(logits - m[:, None]), axis=1)) + m
    infonce = jnp.mean(lse - pos)

    loss_ref[...] = jnp.reshape(mse * (1.0 + _BETA) + infonce, (1, 1))


def kernel(flat_latents, label, embedding):
    q = _sc_gather(embedding, label)
    loss = pl.pallas_call(
        _loss_kernel,
        out_shape=jax.ShapeDtypeStruct((1, 1), jnp.float32),
    )(flat_latents, label.reshape(1, _B), embedding)
    return q, loss.reshape(())


# speedup vs baseline: 1.0155x; 1.0035x over previous
import functools

import jax
import jax.numpy as jnp
from jax import lax
from jax.experimental import pallas as pl
from jax.experimental.pallas import tpu as pltpu
from jax.experimental.pallas import tpu_sc as plsc

_K = 512
_D = 256
_B = 576
_BETA = 0.25
_TEMP = 0.1

_ROWS_PER_WORKER = 48
_NUM_WORKERS = _B // _ROWS_PER_WORKER


def _sc_gather(embedding, label):
    mesh = plsc.VectorSubcoreMesh(
        core_axis_name="c", subcore_axis_name="s", num_cores=1,
        num_subcores=_NUM_WORKERS)

    @functools.partial(
        pl.kernel,
        mesh=mesh,
        out_type=jax.ShapeDtypeStruct((_B, _D), jnp.float32),
        scratch_types=[
            pltpu.VMEM((_ROWS_PER_WORKER,), jnp.int32),
            pltpu.VMEM((_ROWS_PER_WORKER, _D), jnp.float32),
            pltpu.SemaphoreType.DMA,
        ],
    )
    def gather_kernel(table_hbm, idx_hbm, out_hbm, idx_v, rows_v, sem):
        wid = lax.axis_index("s")

        @pl.when(wid < _NUM_WORKERS)
        def _():
            base = wid * _ROWS_PER_WORKER
            pltpu.sync_copy(idx_hbm.at[pl.ds(base, _ROWS_PER_WORKER)], idx_v)
            pltpu.async_copy(table_hbm.at[idx_v], rows_v, sem).wait()
            pltpu.sync_copy(rows_v, out_hbm.at[pl.ds(base, _ROWS_PER_WORKER)])

    return gather_kernel(embedding, label)


def _loss_kernel(x_ref, lab_ref, e_ref, loss_ref):
    x = x_ref[...]
    e = e_ref[...]
    lab = lab_ref[...]

    ohT = (jax.lax.broadcasted_iota(jnp.int32, (_K, _B), 0) == lab).astype(
        jnp.float32)
    p = jax.lax.dot_general(
        ohT, e, (((0,), (0,)), ((), ())),
        preferred_element_type=jnp.float32,
        precision=jax.lax.Precision.HIGHEST)

    mse = jnp.mean((p - x) ** 2)

    qn = x * jax.lax.rsqrt(jnp.sum(x * x, axis=1, keepdims=True))
    en = e * jax.lax.rsqrt(jnp.sum(e * e, axis=1, keepdims=True))
    pn = p * jax.lax.rsqrt(jnp.sum(p * p, axis=1, keepdims=True))

    logits = jax.lax.dot_general(
        qn, en, (((1,), (1,)), ((), ())),
        preferred_element_type=jnp.float32,
        precision=jax.lax.Precision.HIGHEST) * (1.0 / _TEMP)
    pos = jnp.sum(qn * pn, axis=1) * (1.0 / _TEMP)

    m = jnp.max(logits, axis=1)
    lse = jnp.log(jnp.sum(jnp.exp(logits - m[:, None]), axis=1)) + m
    infonce = jnp.mean(lse - pos)

    loss_ref[...] = jnp.reshape(mse * (1.0 + _BETA) + infonce, (1, 1))


def kernel(flat_latents, label, embedding):
    q = _sc_gather(embedding, label)
    loss = pl.pallas_call(
        _loss_kernel,
        out_shape=jax.ShapeDtypeStruct((1, 1), jnp.float32),
    )(flat_latents, label.reshape(1, _B), embedding)
    return q, loss.reshape(())
